# baseline (speedup 1.0000x reference)
import functools

import jax
import jax.numpy as jnp
from jax import lax
from jax.experimental import pallas as pl
from jax.experimental.pallas import tpu as pltpu
from jax.experimental.pallas import tpu_sc as plsc

BATCH, SEQ = 4096, 200
HIDDEN = 128
HALF = HIDDEN // 2
N = BATCH * SEQ
NC, NS = 2, 16
NW = NC * NS
PER_W = N // NW
T = 128
CHUNKS = PER_W // T
L = 16
NBUF = 3
PK = HALF // 2


def _embed_body(i0_hbm, i1_hbm, i2_hbm, x_hbm, y_hbm, tlo_hbm, thi_hbm,
                out_hbm, idx0, idx1, idx2, xbuf, ybuf, tlobuf, thibuf,
                obuf, ssem, gsem, osem):
    wid = lax.axis_index("s") * NC + lax.axis_index("c")
    base = wid * PER_W

    def stage(c):
        p = c % NBUF
        src = pl.ds(base + c * T, T)
        pltpu.async_copy(i0_hbm.at[src], idx0.at[p], ssem.at[p])
        pltpu.async_copy(i1_hbm.at[src], idx1.at[p], ssem.at[p])
        pltpu.async_copy(i2_hbm.at[src], idx2.at[p], ssem.at[p])

    def wait_stage(p):
        dummy = pl.ds(0, T)
        pltpu.make_async_copy(i0_hbm.at[dummy], idx0.at[p], ssem.at[p]).wait()
        pltpu.make_async_copy(i1_hbm.at[dummy], idx1.at[p], ssem.at[p]).wait()
        pltpu.make_async_copy(i2_hbm.at[dummy], idx2.at[p], ssem.at[p]).wait()

    def bump(p):
        def bbody(i, carry):
            s = pl.ds(i * L, L)
            idx0[p, s] = idx0[p, s] + 1
            idx1[p, s] = idx1[p, s] + 1
            idx2[p, s] = idx2[p, s] + 1
            return carry
        lax.fori_loop(0, T // L, bbody, 0, unroll=True)

    def fire_gathers(c):
        p = c % NBUF
        pltpu.async_copy(x_hbm.at[idx0.at[p]], xbuf.at[p], gsem.at[p])
        pltpu.async_copy(y_hbm.at[idx1.at[p]], ybuf.at[p], gsem.at[p])
        pltpu.async_copy(tlo_hbm.at[idx2.at[p]], tlobuf.at[p], gsem.at[p])
        pltpu.async_copy(thi_hbm.at[idx2.at[p]], thibuf.at[p], gsem.at[p])

    def wait_gathers(p):
        pltpu.make_async_copy(x_hbm.at[idx0.at[p]], xbuf.at[p],
                              gsem.at[p]).wait()
        pltpu.make_async_copy(y_hbm.at[idx1.at[p]], ybuf.at[p],
                              gsem.at[p]).wait()
        pltpu.make_async_copy(tlo_hbm.at[idx2.at[p]], tlobuf.at[p],
                              gsem.at[p]).wait()
        pltpu.make_async_copy(thi_hbm.at[idx2.at[p]], thibuf.at[p],
                              gsem.at[p]).wait()

    def fire_store(c):
        p = c % NBUF
        pltpu.async_copy(obuf.at[p], out_hbm.at[pl.ds(base + c * T, T)],
                         osem.at[p])

    def wait_store(p):
        pltpu.make_async_copy(obuf.at[p], out_hbm.at[pl.ds(base, T)],
                              osem.at[p]).wait()

    def combine(q):
        msk = jnp.int32(-65536)

        def fp(v):
            return jax.lax.bitcast_convert_type(v, jnp.float32)

        def vbody(t, carry):
            for j in range(2):
                s = pl.ds(L * j, L)
                xv = xbuf[q, t, s]
                tv = tlobuf[q, t, s]
                obuf[q, t, pl.ds(32 * j, L)] = fp(xv << 16) + fp(tv << 16)
                obuf[q, t, pl.ds(32 * j + L, L)] = fp(xv & msk) + fp(tv & msk)
                yv = ybuf[q, t, s]
                hv = thibuf[q, t, s]
                obuf[q, t, pl.ds(HALF + 32 * j, L)] = (fp(yv << 16) +
                                                       fp(hv << 16))
                obuf[q, t, pl.ds(HALF + 32 * j + L, L)] = (fp(yv & msk) +
                                                           fp(hv & msk))
            return carry

        lax.fori_loop(0, T, vbody, 0)

    stage(0)
    stage(1)

    def it(c, carry):
        @pl.when(jnp.logical_and(c >= 1, c <= CHUNKS))
        def _back():
            q = (c - 1) % NBUF
            wait_gathers(q)

            @pl.when(c - 1 >= NBUF)
            def _reuse():
                wait_store(q)

            combine(q)
            fire_store(c - 1)

        @pl.when(c < CHUNKS)
        def _front():
            p = c % NBUF
            wait_stage(p)
            bump(p)
            fire_gathers(c)

            @pl.when(c + 2 < CHUNKS)
            def _stage_ahead():
                stage(c + 2)

        return carry

    lax.fori_loop(0, CHUNKS + 1, it, 0)

    for k in range(CHUNKS - NBUF, CHUNKS):
        wait_store(k % NBUF)


@jax.jit
def _embed(i0, i1, i2, x_p, y_p, tlo_p, thi_p):
    mesh = plsc.VectorSubcoreMesh(core_axis_name="c", subcore_axis_name="s",
                                  num_cores=NC, num_subcores=NS)
    f = pl.kernel(
        _embed_body,
        out_type=jax.ShapeDtypeStruct((N, HIDDEN), jnp.float32),
        mesh=mesh,
        compiler_params=pltpu.CompilerParams(use_tc_tiling_on_sc=False),
        scratch_types=[
            pltpu.VMEM((NBUF, T), jnp.int32),
            pltpu.VMEM((NBUF, T), jnp.int32),
            pltpu.VMEM((NBUF, T), jnp.int32),
            pltpu.VMEM((NBUF, T, PK), jnp.int32),
            pltpu.VMEM((NBUF, T, PK), jnp.int32),
            pltpu.VMEM((NBUF, T, PK), jnp.int32),
            pltpu.VMEM((NBUF, T, PK), jnp.int32),
            pltpu.VMEM((NBUF, T, HIDDEN), jnp.float32),
            pltpu.SemaphoreType.DMA((NBUF,)),
            pltpu.SemaphoreType.DMA((NBUF,)),
            pltpu.SemaphoreType.DMA((NBUF,)),
        ],
    )
    return f(i0, i1, i2, x_p, y_p, tlo_p, thi_p)


def _packcols(tbl):
    v, w = tbl.shape
    nb = w // 32
    t4 = tbl.reshape(v, nb, 2, L).transpose(0, 1, 3, 2)
    tb = t4.astype(jnp.bfloat16)
    ti = jax.lax.bitcast_convert_type(tb, jnp.int32)
    return ti.reshape(v, w // 2)


def kernel(input_states, x_embedding, y_embedding, type_embedding):
    inp = input_states.reshape(N, 3).astype(jnp.int32)
    i0 = inp[:, 0]
    i1 = inp[:, 1]
    i2 = inp[:, 2]
    x_p = _packcols(x_embedding)
    y_p = _packcols(y_embedding)
    tlo_p = _packcols(type_embedding[:, :HALF])
    thi_p = _packcols(type_embedding[:, HALF:])
    out = _embed(i0, i1, i2, x_p, y_p, tlo_p, thi_p)
    return out.reshape(BATCH, SEQ, HIDDEN)


# speedup vs baseline: 1.0528x; 1.0528x over previous
import functools

import jax
import jax.numpy as jnp
from jax import lax
from jax.experimental import pallas as pl
from jax.experimental.pallas import tpu as pltpu
from jax.experimental.pallas import tpu_sc as plsc

BATCH, SEQ = 4096, 200
HIDDEN = 128
HALF = HIDDEN // 2
N = BATCH * SEQ
NC, NS = 2, 16
NW = NC * NS
PER_W = N // NW
T = 128
CHUNKS = PER_W // T
L = 16
NBUF = 3
PK = HALF // 2


def _embed_body(i0_hbm, i1_hbm, i2_hbm, x_hbm, y_hbm, tlo_hbm, thi_hbm,
                out_hbm, idx0, idx1, idx2, xbuf, ybuf, tlobuf, thibuf,
                obuf, ssem, gsem, osem):
    wid = lax.axis_index("s") * NC + lax.axis_index("c")
    base = wid * PER_W

    def stage(c):
        p = c % NBUF
        src = pl.ds(base + c * T, T)
        pltpu.async_copy(i0_hbm.at[src], idx0.at[p], ssem.at[p])
        pltpu.async_copy(i1_hbm.at[src], idx1.at[p], ssem.at[p])
        pltpu.async_copy(i2_hbm.at[src], idx2.at[p], ssem.at[p])

    def wait_stage(p):
        dummy = pl.ds(0, T)
        pltpu.make_async_copy(i0_hbm.at[dummy], idx0.at[p], ssem.at[p]).wait()
        pltpu.make_async_copy(i1_hbm.at[dummy], idx1.at[p], ssem.at[p]).wait()
        pltpu.make_async_copy(i2_hbm.at[dummy], idx2.at[p], ssem.at[p]).wait()

    def bump(p):
        def bbody(i, carry):
            s = pl.ds(i * L, L)
            idx0[p, s] = idx0[p, s] + 1
            idx1[p, s] = idx1[p, s] + 1
            idx2[p, s] = idx2[p, s] + 1
            return carry
        lax.fori_loop(0, T // L, bbody, 0, unroll=True)

    def fire_gathers(c):
        p = c % NBUF
        pltpu.async_copy(x_hbm.at[idx0.at[p]], xbuf.at[p], gsem.at[p])
        pltpu.async_copy(y_hbm.at[idx1.at[p]], ybuf.at[p], gsem.at[p])
        pltpu.async_copy(tlo_hbm.at[idx2.at[p]], tlobuf.at[p], gsem.at[p])
        pltpu.async_copy(thi_hbm.at[idx2.at[p]], thibuf.at[p], gsem.at[p])

    def wait_gathers(p):
        pltpu.make_async_copy(x_hbm.at[idx0.at[p]], xbuf.at[p],
                              gsem.at[p]).wait()
        pltpu.make_async_copy(y_hbm.at[idx1.at[p]], ybuf.at[p],
                              gsem.at[p]).wait()
        pltpu.make_async_copy(tlo_hbm.at[idx2.at[p]], tlobuf.at[p],
                              gsem.at[p]).wait()
        pltpu.make_async_copy(thi_hbm.at[idx2.at[p]], thibuf.at[p],
                              gsem.at[p]).wait()

    def fire_store(c):
        p = c % NBUF
        pltpu.async_copy(obuf.at[p], out_hbm.at[pl.ds(base + c * T, T)],
                         osem.at[p])

    def wait_store(p):
        pltpu.make_async_copy(obuf.at[p], out_hbm.at[pl.ds(base, T)],
                              osem.at[p]).wait()

    def combine(q):
        msk = jnp.int32(-65536)

        def fp(v):
            return jax.lax.bitcast_convert_type(v, jnp.float32)

        def vbody(t, carry):
            for j in range(2):
                s = pl.ds(L * j, L)
                xv = xbuf[q, t, s]
                tv = tlobuf[q, t, s]
                obuf[q, t, pl.ds(32 * j, L)] = fp(xv << 16) + fp(tv << 16)
                obuf[q, t, pl.ds(32 * j + L, L)] = fp(xv & msk) + fp(tv & msk)
                yv = ybuf[q, t, s]
                hv = thibuf[q, t, s]
                obuf[q, t, pl.ds(HALF + 32 * j, L)] = (fp(yv << 16) +
                                                       fp(hv << 16))
                obuf[q, t, pl.ds(HALF + 32 * j + L, L)] = (fp(yv & msk) +
                                                           fp(hv & msk))
            return carry

        lax.fori_loop(0, T, vbody, 0, unroll=4)

    stage(0)
    stage(1)

    def it(c, carry):
        @pl.when(jnp.logical_and(c >= 1, c <= CHUNKS))
        def _back():
            q = (c - 1) % NBUF
            wait_gathers(q)

            @pl.when(c - 1 >= NBUF)
            def _reuse():
                wait_store(q)

            combine(q)
            fire_store(c - 1)

        @pl.when(c < CHUNKS)
        def _front():
            p = c % NBUF
            wait_stage(p)
            bump(p)
            fire_gathers(c)

            @pl.when(c + 2 < CHUNKS)
            def _stage_ahead():
                stage(c + 2)

        return carry

    lax.fori_loop(0, CHUNKS + 1, it, 0)

    for k in range(CHUNKS - NBUF, CHUNKS):
        wait_store(k % NBUF)


@jax.jit
def _embed(i0, i1, i2, x_p, y_p, tlo_p, thi_p):
    mesh = plsc.VectorSubcoreMesh(core_axis_name="c", subcore_axis_name="s",
                                  num_cores=NC, num_subcores=NS)
    f = pl.kernel(
        _embed_body,
        out_type=jax.ShapeDtypeStruct((N, HIDDEN), jnp.float32),
        mesh=mesh,
        compiler_params=pltpu.CompilerParams(use_tc_tiling_on_sc=False),
        scratch_types=[
            pltpu.VMEM((NBUF, T), jnp.int32),
            pltpu.VMEM((NBUF, T), jnp.int32),
            pltpu.VMEM((NBUF, T), jnp.int32),
            pltpu.VMEM((NBUF, T, PK), jnp.int32),
            pltpu.VMEM((NBUF, T, PK), jnp.int32),
            pltpu.VMEM((NBUF, T, PK), jnp.int32),
            pltpu.VMEM((NBUF, T, PK), jnp.int32),
            pltpu.VMEM((NBUF, T, HIDDEN), jnp.float32),
            pltpu.SemaphoreType.DMA((NBUF,)),
            pltpu.SemaphoreType.DMA((NBUF,)),
            pltpu.SemaphoreType.DMA((NBUF,)),
        ],
    )
    return f(i0, i1, i2, x_p, y_p, tlo_p, thi_p)


def _packcols(tbl):
    v, w = tbl.shape
    nb = w // 32
    t4 = tbl.reshape(v, nb, 2, L).transpose(0, 1, 3, 2)
    tb = t4.astype(jnp.bfloat16)
    ti = jax.lax.bitcast_convert_type(tb, jnp.int32)
    return ti.reshape(v, w // 2)


def kernel(input_states, x_embedding, y_embedding, type_embedding):
    inp = input_states.reshape(N, 3).astype(jnp.int32)
    i0 = inp[:, 0]
    i1 = inp[:, 1]
    i2 = inp[:, 2]
    x_p = _packcols(x_embedding)
    y_p = _packcols(y_embedding)
    tlo_p = _packcols(type_embedding[:, :HALF])
    thi_p = _packcols(type_embedding[:, HALF:])
    out = _embed(i0, i1, i2, x_p, y_p, tlo_p, thi_p)
    return out.reshape(BATCH, SEQ, HIDDEN)
